# baseline (speedup 1.0000x reference)
import functools

import jax
import jax.numpy as jnp
from jax import lax
from jax.experimental import pallas as pl
from jax.experimental.pallas import tpu as pltpu
from jax.experimental.pallas import tpu_sc as plsc


def _sc_gather(table, idx, n_frames, dim):
    n_workers = 8
    rows_per_w = n_frames // n_workers

    mesh = plsc.VectorSubcoreMesh(core_axis_name="c", subcore_axis_name="s")

    @functools.partial(
        pl.kernel,
        mesh=mesh,
        out_type=jax.ShapeDtypeStruct((n_frames, dim), jnp.float32),
        scratch_types=[
            pltpu.VMEM((rows_per_w,), jnp.int32),
            pltpu.VMEM((rows_per_w, dim), jnp.float32),
            pltpu.SemaphoreType.DMA,
        ],
    )
    def gather_kernel(table_hbm, idx_hbm, out_hbm, idx_v, rows_v, sem):
        wid = lax.axis_index("s") * 2 + lax.axis_index("c")

        @pl.when(wid < n_workers)
        def _():
            base = wid * rows_per_w
            pltpu.sync_copy(idx_hbm.at[pl.ds(base, rows_per_w)], idx_v)
            pltpu.async_copy(table_hbm.at[idx_v], rows_v, sem).wait()
            pltpu.sync_copy(rows_v, out_hbm.at[pl.ds(base, rows_per_w)])

    return gather_kernel(table, idx)


def _tc_broadcast_body(colmat_ref, out_ref):
    i = pl.program_id(0)
    dim, nf = colmat_ref.shape
    hw = out_ref.shape[2]
    sel = lax.broadcasted_iota(jnp.int32, (dim, nf), 1) == i
    col = jnp.sum(jnp.where(sel, colmat_ref[...], 0.0), axis=1, keepdims=True)
    out_ref[...] = jnp.broadcast_to(col.reshape(1, dim, 1), (1, dim, hw))


def kernel(spatialPos, numFrames, frameIndices, frameEmbed):
    _, _, height, width = spatialPos.shape
    n_frames = frameIndices.shape[0]
    dim = frameEmbed.shape[1]
    hw = height * width

    gathered = _sc_gather(frameEmbed, frameIndices.astype(jnp.int32),
                          n_frames, dim)
    gathered_t = gathered.T

    out = pl.pallas_call(
        _tc_broadcast_body,
        grid=(n_frames,),
        in_specs=[pl.BlockSpec((dim, n_frames), lambda i: (0, 0))],
        out_specs=pl.BlockSpec((1, dim, hw), lambda i: (i, 0, 0)),
        out_shape=jax.ShapeDtypeStruct((n_frames, dim, hw), jnp.float32),
    )(gathered_t)

    return out.reshape(n_frames, dim, height, width)


# speedup vs baseline: 1.2242x; 1.2242x over previous
import jax
import jax.numpy as jnp
from jax import lax
from jax.experimental import pallas as pl
from jax.experimental.pallas import tpu as pltpu


def _body(idx_ref, tbl_ref, out_ref):
    i = pl.program_id(0)
    dim, vocab = tbl_ref.shape
    hw = out_ref.shape[2]
    v = idx_ref[i]
    sel = lax.broadcasted_iota(jnp.int32, (dim, vocab), 1) == v
    col = jnp.sum(jnp.where(sel, tbl_ref[...], 0.0), axis=1, keepdims=True)
    out_ref[...] = jnp.broadcast_to(col.reshape(1, dim, 1), (1, dim, hw))


def kernel(spatialPos, numFrames, frameIndices, frameEmbed):
    _, _, height, width = spatialPos.shape
    n_frames = frameIndices.shape[0]
    vocab, dim = frameEmbed.shape
    hw = height * width

    grid_spec = pltpu.PrefetchScalarGridSpec(
        num_scalar_prefetch=1,
        grid=(n_frames,),
        in_specs=[pl.BlockSpec((dim, vocab), lambda i, s: (0, 0))],
        out_specs=pl.BlockSpec((1, dim, hw), lambda i, s: (i, 0, 0)),
    )
    out = pl.pallas_call(
        _body,
        grid_spec=grid_spec,
        out_shape=jax.ShapeDtypeStruct((n_frames, dim, hw), jnp.float32),
    )(frameIndices.astype(jnp.int32), frameEmbed.T)

    return out.reshape(n_frames, dim, height, width)


# speedup vs baseline: 1.4103x; 1.1520x over previous
import jax
import jax.numpy as jnp
from jax import lax
from jax.experimental import pallas as pl
from jax.experimental.pallas import tpu as pltpu


def _body(idx_ref, tbl_ref, out_ref):
    i = pl.program_id(0)
    dim, vocab = tbl_ref.shape
    bf, _, hw = out_ref.shape
    for j in range(bf):
        v = idx_ref[i * bf + j]
        sel = lax.broadcasted_iota(jnp.int32, (dim, vocab), 1) == v
        col = jnp.sum(jnp.where(sel, tbl_ref[...], 0.0), axis=1, keepdims=True)
        out_ref[j] = jnp.broadcast_to(col, (dim, hw))


def kernel(spatialPos, numFrames, frameIndices, frameEmbed):
    _, _, height, width = spatialPos.shape
    n_frames = frameIndices.shape[0]
    vocab, dim = frameEmbed.shape
    hw = height * width

    bf = 4
    grid_spec = pltpu.PrefetchScalarGridSpec(
        num_scalar_prefetch=1,
        grid=(n_frames // bf,),
        in_specs=[pl.BlockSpec((dim, vocab), lambda i, s: (0, 0))],
        out_specs=pl.BlockSpec((bf, dim, hw), lambda i, s: (i, 0, 0)),
    )
    out = pl.pallas_call(
        _body,
        grid_spec=grid_spec,
        out_shape=jax.ShapeDtypeStruct((n_frames, dim, hw), jnp.float32),
    )(frameIndices.astype(jnp.int32), frameEmbed.T)

    return out.reshape(n_frames, dim, height, width)


# speedup vs baseline: 1.4282x; 1.0127x over previous
import jax
import jax.numpy as jnp
from jax import lax
from jax.experimental import pallas as pl
from jax.experimental.pallas import tpu as pltpu

_NBUF = 4


def _body(idx_ref, tbl_ref, out_ref, buf_ref, sem_ref):
    i = pl.program_id(0)
    n = pl.num_programs(0)
    dim, vocab = tbl_ref.shape
    hw = buf_ref.shape[2]
    s = lax.rem(i, _NBUF)

    @pl.when(i >= _NBUF)
    def _():
        pltpu.make_async_copy(buf_ref.at[s], out_ref.at[i - _NBUF], sem_ref.at[s]).wait()

    v = idx_ref[i]
    sel = lax.broadcasted_iota(jnp.int32, (dim, vocab), 1) == v
    col = jnp.sum(jnp.where(sel, tbl_ref[...], 0.0), axis=1, keepdims=True)
    buf_ref[s] = jnp.broadcast_to(col, (dim, hw))
    pltpu.make_async_copy(buf_ref.at[s], out_ref.at[i], sem_ref.at[s]).start()

    @pl.when(i == n - 1)
    def _():
        for k in range(_NBUF):
            step = n - _NBUF + k
            slot = step % _NBUF
            pltpu.make_async_copy(
                buf_ref.at[slot], out_ref.at[step], sem_ref.at[slot]
            ).wait()


def kernel(spatialPos, numFrames, frameIndices, frameEmbed):
    _, _, height, width = spatialPos.shape
    n_frames = frameIndices.shape[0]
    vocab, dim = frameEmbed.shape
    hw = height * width

    grid_spec = pltpu.PrefetchScalarGridSpec(
        num_scalar_prefetch=1,
        grid=(n_frames,),
        in_specs=[pl.BlockSpec((dim, vocab), lambda i, s: (0, 0))],
        out_specs=pl.BlockSpec(memory_space=pltpu.MemorySpace.HBM),
        scratch_shapes=[
            pltpu.VMEM((_NBUF, dim, hw), jnp.float32),
            pltpu.SemaphoreType.DMA((_NBUF,)),
        ],
    )
    out = pl.pallas_call(
        _body,
        grid_spec=grid_spec,
        out_shape=jax.ShapeDtypeStruct((n_frames, dim, hw), jnp.float32),
    )(frameIndices.astype(jnp.int32), frameEmbed.T)

    return out.reshape(n_frames, dim, height, width)
